# baseline (speedup 1.0000x reference)
import functools

import jax
import jax.numpy as jnp
from jax import lax
from jax.experimental import pallas as pl
from jax.experimental.pallas import tpu as pltpu
from jax.experimental.pallas import tpu_sc as plsc

ACQ = 1
B, A, S, C = 64, 4, 4096, 2
ROW = S * C

_NC = 2
_NS = 16
_ROWS_PER_W = B // (_NC * _NS)


def _copy_body(in_hbm, out_hbm, sem0, sem1):
    wid = lax.axis_index("s") * _NC + lax.axis_index("c")
    b = wid * _ROWS_PER_W
    c0 = pltpu.make_async_copy(in_hbm.at[A * b + ACQ], out_hbm.at[b], sem0)
    c1 = pltpu.make_async_copy(
        in_hbm.at[A * (b + 1) + ACQ], out_hbm.at[b + 1], sem1
    )
    c0.start()
    c1.start()
    c0.wait()
    c1.wait()


_copy = functools.partial(
    pl.kernel,
    out_type=jax.ShapeDtypeStruct((B, ROW), jnp.float32),
    mesh=plsc.VectorSubcoreMesh(core_axis_name="c", subcore_axis_name="s"),
    scratch_types=[pltpu.SemaphoreType.DMA, pltpu.SemaphoreType.DMA],
)(_copy_body)


@jax.jit
def kernel(inputs):
    x = inputs.reshape(B * A, ROW)
    out = _copy(x)
    return out.reshape(B, S, C)
